# baseline (speedup 1.0000x reference)
import functools

import jax
import jax.numpy as jnp
from jax import lax
from jax.experimental import pallas as pl
from jax.experimental.pallas import tpu as pltpu
from jax.experimental.pallas import tpu_sc as plsc

N = 50000
E = 800000
D = 64
R = 8

NS = 16
NC = 2
BATCH = 96
NBB = 2
MACRO = BATCH * NBB
NB = 268
EP = NB * MACRO
E2 = EP * NS
NRP = 400384
NHALF = N // 2
ACC = 25088
DUMP = NHALF
BLK = 1000
CBLK = 200

_mesh = plsc.VectorSubcoreMesh(core_axis_name="c", subcore_axis_name="s")
_params = pltpu.CompilerParams(needs_layout_passes=False,
                               use_tc_tiling_on_sc=False)


def _fill(ref, n, val):
    v = jnp.full((16,), val, ref.dtype)
    def body(i, carry):
        ref[pl.ds(i * 16, 16)] = v
        return carry
    lax.fori_loop(0, n // 16, body, 0)


def _hist_body(dst_h, et_h, out_h, cnt_sh, dbuf, tbuf, segbuf, ones, zbuf):
    c = lax.axis_index("c")
    s = lax.axis_index("s")
    _fill(zbuf, 3200, 0.0)
    _fill(ones, BATCH, 1.0)
    for j in range(8):
        pltpu.sync_copy(zbuf.at[pl.ds(0, 3128)],
                        cnt_sh.at[pl.ds(s * 25024 + j * 3128, 3128)])
    plsc.subcore_barrier()

    def macro_body(k, carry):
        base = s * EP + k * MACRO
        pltpu.sync_copy(dst_h.at[pl.ds(base, MACRO)], dbuf)
        pltpu.sync_copy(et_h.at[pl.ds(base, MACRO)], tbuf)
        for b in range(NBB):
            for j in range(BATCH // 16):
                o = b * BATCH + j * 16
                dv = dbuf[pl.ds(o, 16)]
                tv = tbuf[pl.ds(o, 16)]
                segbuf[pl.ds(j * 16, 16)] = dv * R + tv
            pltpu.sync_copy(ones, cnt_sh.at[segbuf], add=True)
        return carry

    lax.fori_loop(0, NB, macro_body, 0)
    plsc.subcore_barrier()
    half = NRP // 2
    for j in range(4):
        off = c * half + s * 12512 + j * 3128
        pltpu.sync_copy(cnt_sh.at[pl.ds(off, 3128)], zbuf.at[pl.ds(0, 3128)])
        pltpu.sync_copy(zbuf.at[pl.ds(0, 3128)], out_h.at[pl.ds(off, 3128)])


_hist = functools.partial(
    pl.kernel,
    out_type=jax.ShapeDtypeStruct((NRP,), jnp.float32),
    mesh=_mesh,
    compiler_params=_params,
    scratch_types=[
        pltpu.VMEM_SHARED((NRP,), jnp.float32),
        pltpu.VMEM((MACRO,), jnp.int32),
        pltpu.VMEM((MACRO,), jnp.int32),
        pltpu.VMEM((BATCH,), jnp.int32),
        pltpu.VMEM((BATCH,), jnp.float32),
        pltpu.VMEM((3200,), jnp.float32),
    ],
)(_hist_body)


WEP = E2 // 32
WNB = WEP // MACRO


def _wk_body(dst_h, et_h, cnt_h, out_h, dbuf, tbuf, segbuf, cbuf, wbuf, gsem):
    c = lax.axis_index("c")
    s = lax.axis_index("s")
    wid = c * NS + s

    def macro_body(k, carry):
        base = wid * WEP + k * MACRO
        pltpu.sync_copy(dst_h.at[pl.ds(base, MACRO)], dbuf)
        pltpu.sync_copy(et_h.at[pl.ds(base, MACRO)], tbuf)
        for b in range(NBB):
            for j in range(BATCH // 16):
                o = b * BATCH + j * 16
                dv = dbuf[pl.ds(o, 16)]
                tv = tbuf[pl.ds(o, 16)]
                segbuf[b, pl.ds(j * 16, 16)] = dv * R + tv
        descs = [pltpu.async_copy(cnt_h.at[segbuf.at[b]],
                                  cbuf.at[pl.ds(b * BATCH, BATCH)], gsem)
                 for b in range(NBB)]
        for d in descs:
            d.wait()
        for j in range(MACRO // 16):
            cv = cbuf[pl.ds(j * 16, 16)]
            wbuf[pl.ds(j * 16, 16)] = 1.0 / jnp.maximum(cv, 1.0)
        pltpu.sync_copy(wbuf, out_h.at[pl.ds(base, MACRO)])
        return carry

    lax.fori_loop(0, WNB, macro_body, 0)


_wk = functools.partial(
    pl.kernel,
    out_type=jax.ShapeDtypeStruct((E2,), jnp.float32),
    mesh=_mesh,
    compiler_params=_params,
    scratch_types=[
        pltpu.VMEM((MACRO,), jnp.int32),
        pltpu.VMEM((MACRO,), jnp.int32),
        pltpu.VMEM((NBB, BATCH), jnp.int32),
        pltpu.VMEM((MACRO,), jnp.float32),
        pltpu.VMEM((MACRO,), jnp.float32),
        pltpu.SemaphoreType.DMA,
    ],
)(_wk_body)


def _edge_body(src_h, et_h, dst_h, w_h, hflat_h, out_h,
               acc_sh, sbuf, tbuf, dbuf, wsbuf, gbuf, lbuf, wm, rows,
               gsem, stsem, ssem0, ssem1, ssem2, ssem3):
    c = lax.axis_index("c")
    s = lax.axis_index("s")
    lo = c * NHALF
    ssems = (ssem0, ssem1, ssem2, ssem3)

    def zfill(i, carry):
        for kk in range(4):
            rows[i, pl.ds(kk * 16, 16)] = jnp.zeros((16,), jnp.float32)
        return carry
    lax.fori_loop(0, BATCH, zfill, 0)
    for j in range(16):
        pltpu.sync_copy(rows.at[pl.ds(0, 96)],
                        acc_sh.at[pl.ds(s * 1568 + j * 96, 96)])
    pltpu.sync_copy(rows.at[pl.ds(0, 32)],
                    acc_sh.at[pl.ds(s * 1568 + 1536, 32)])
    plsc.subcore_barrier()

    def _stage(m, q, issue):
        base = s * EP + m * MACRO
        f = pltpu.async_copy if issue else pltpu.make_async_copy
        return [
            f(src_h.at[pl.ds(base, MACRO)], sbuf.at[q], stsem),
            f(et_h.at[pl.ds(base, MACRO)], tbuf.at[q], stsem),
            f(dst_h.at[pl.ds(base, MACRO)], dbuf.at[q], stsem),
            f(w_h.at[pl.ds(base, MACRO)], wsbuf.at[q], stsem),
        ]

    _stage(0, 0, True)

    def _macro(mm, mq):
        m = 2 * mm + mq
        for d in _stage(m, mq, False):
            d.wait()
        for b in range(NBB):
            for j in range(BATCH // 16):
                o = b * BATCH + j * 16
                sv = sbuf[mq, pl.ds(o, 16)]
                tv = tbuf[mq, pl.ds(o, 16)]
                dv = dbuf[mq, pl.ds(o, 16)]
                wv = wsbuf[mq, pl.ds(o, 16)]
                gbuf[mq, b, pl.ds(j * 16, 16)] = tv * N + sv
                m_in = (dv >= lo) & (dv < lo + NHALF)
                lbuf[mq, b, pl.ds(j * 16, 16)] = jnp.where(m_in, dv - lo, DUMP)
                wm[mq, pl.ds(o, 16)] = jnp.where(m_in, wv, 0.0)
        gds = []
        for b in range(NBB):
            slot = 2 * mq + b
            sl = slot * BATCH

            @pl.when(m >= 2)
            def _():
                pltpu.make_async_copy(rows.at[pl.ds(sl, BATCH)],
                                      acc_sh.at[lbuf.at[0, 0]],
                                      ssems[slot]).wait()
            gds.append(pltpu.async_copy(hflat_h.at[gbuf.at[mq, b]],
                                        rows.at[pl.ds(sl, BATCH)], gsem))

        @pl.when(m < NB - 1)
        def _():
            _stage(m + 1, 1 - mq, True)

        for b in range(NBB):
            slot = 2 * mq + b
            sl = slot * BATCH
            gds[b].wait()
            wmv = wm.at[mq]

            def mul_body(i, carry2):
                for u in range(4):
                    ri = sl + i * 4 + u
                    wv = plsc.load_gather(
                        wmv, [jnp.full((16,), b * BATCH + i * 4 + u,
                                       jnp.int32)])
                    for kk in range(4):
                        rows[ri, pl.ds(kk * 16, 16)] = (
                            rows[ri, pl.ds(kk * 16, 16)] * wv)
                return carry2
            lax.fori_loop(0, BATCH // 4, mul_body, 0)
            pltpu.async_copy(rows.at[pl.ds(sl, BATCH)],
                             acc_sh.at[lbuf.at[mq, b]], ssems[slot], add=True)

    def mm_body(mm, carry):
        _macro(mm, 0)
        _macro(mm, 1)
        return carry

    lax.fori_loop(0, NB // 2, mm_body, 0)
    for slot in range(4):
        pltpu.make_async_copy(rows.at[pl.ds(slot * BATCH, BATCH)],
                              acc_sh.at[lbuf.at[0, 0]], ssems[slot]).wait()
    plsc.subcore_barrier()

    for j in range(16):
        pltpu.sync_copy(acc_sh.at[pl.ds(s * 1568 + j * 96, 96)],
                        rows.at[pl.ds(0, 96)])
        pltpu.sync_copy(rows.at[pl.ds(0, 96)],
                        out_h.at[c, pl.ds(s * 1568 + j * 96, 96)])
    pltpu.sync_copy(acc_sh.at[pl.ds(s * 1568 + 1536, 32)],
                    rows.at[pl.ds(0, 32)])
    pltpu.sync_copy(rows.at[pl.ds(0, 32)],
                    out_h.at[c, pl.ds(s * 1568 + 1536, 32)])


_edge = functools.partial(
    pl.kernel,
    out_type=jax.ShapeDtypeStruct((2, ACC, D), jnp.float32),
    mesh=_mesh,
    compiler_params=_params,
    scratch_types=[
        pltpu.VMEM_SHARED((ACC, D), jnp.float32),
        pltpu.VMEM((2, MACRO), jnp.int32),
        pltpu.VMEM((2, MACRO), jnp.int32),
        pltpu.VMEM((2, MACRO), jnp.int32),
        pltpu.VMEM((2, MACRO), jnp.float32),
        pltpu.VMEM((2, NBB, BATCH), jnp.int32),
        pltpu.VMEM((2, NBB, BATCH), jnp.int32),
        pltpu.VMEM((2, MACRO), jnp.float32),
        pltpu.VMEM((4 * BATCH, D), jnp.float32),
        pltpu.SemaphoreType.DMA,
        pltpu.SemaphoreType.DMA,
        pltpu.SemaphoreType.DMA,
        pltpu.SemaphoreType.DMA,
        pltpu.SemaphoreType.DMA,
        pltpu.SemaphoreType.DMA,
    ],
)(_edge_body)


def _mm_body(h_ref, w_ref, o_ref):
    h = h_ref[...]
    for r in range(R + 1):
        o_ref[r] = jnp.dot(h, w_ref[r], preferred_element_type=jnp.float32)


def _mm(h, wall):
    return pl.pallas_call(
        _mm_body,
        grid=(N // BLK,),
        in_specs=[
            pl.BlockSpec((BLK, D), lambda i: (i, 0)),
            pl.BlockSpec((R + 1, D, D), lambda i: (0, 0, 0)),
        ],
        out_specs=pl.BlockSpec((R + 1, BLK, D), lambda i: (0, i, 0)),
        out_shape=jax.ShapeDtypeStruct((R + 1, N, D), jnp.float32),
    )(h, wall)


def _combine_body_relu(a_ref, r_ref, b_ref, o_ref):
    o_ref[...] = jnp.maximum(a_ref[0] + r_ref[0] + b_ref[...], 0.0)


def _combine_body(a_ref, r_ref, b_ref, o_ref):
    o_ref[...] = a_ref[0] + r_ref[0] + b_ref[...]


def _combine(aggp, hall, b, relu):
    body = _combine_body_relu if relu else _combine_body
    return pl.pallas_call(
        body,
        grid=(N // CBLK,),
        in_specs=[
            pl.BlockSpec((1, CBLK, D),
                         lambda i: (jnp.where(i < 125, 0, 1),
                                    jnp.where(i < 125, i, i - 125), 0)),
            pl.BlockSpec((1, CBLK, D), lambda i: (R, i, 0)),
            pl.BlockSpec((1, D), lambda i: (0, 0)),
        ],
        out_specs=pl.BlockSpec((CBLK, D), lambda i: (i, 0)),
        out_shape=jax.ShapeDtypeStruct((N, D), jnp.float32),
    )(aggp, hall, b)


def kernel(x, edge_index, edge_type, emb, W1, root1, b1, W2, root2, b2):
    src = edge_index[0].astype(jnp.int32)
    dst = edge_index[1].astype(jnp.int32)
    et = edge_type.astype(jnp.int32)
    pad = E2 - E
    srcp = jnp.concatenate([src, jnp.zeros((pad,), jnp.int32)])
    etp = jnp.concatenate([et, jnp.zeros((pad,), jnp.int32)])
    dstp = jnp.concatenate([dst, jnp.full((pad,), N, jnp.int32)])
    h = jnp.take(emb, x, axis=0)
    cnt = _hist(dstp, etp)
    w = _wk(dstp, etp, cnt)
    for (W, root, b, relu) in ((W1, root1, b1, True), (W2, root2, b2, False)):
        wall = jnp.concatenate([W, root[None]], axis=0)
        hall = _mm(h, wall)
        aggp = _edge(srcp, etp, dstp, w, hall.reshape((R + 1) * N, D))
        h = _combine(aggp, hall, b.reshape(1, D), relu)
    return h


# speedup vs baseline: 1.1782x; 1.1782x over previous
import functools

import jax
import jax.numpy as jnp
from jax import lax
from jax.experimental import pallas as pl
from jax.experimental.pallas import tpu as pltpu
from jax.experimental.pallas import tpu_sc as plsc

N = 50000
E = 800000
D = 64
HD = D // 2
R = 8

NS = 16
NC = 2
BATCH = 128
NBB = 2
MACRO = BATCH * NBB
NB = 202
EP = NB * MACRO
E2 = EP * NS
NRP = 400384
ACC = 50176
BLK = 1000
CBLK = 200

_mesh = plsc.VectorSubcoreMesh(core_axis_name="c", subcore_axis_name="s")
_params = pltpu.CompilerParams(needs_layout_passes=False,
                               use_tc_tiling_on_sc=False)


def _fill(ref, n, val):
    v = jnp.full((16,), val, ref.dtype)
    def body(i, carry):
        ref[pl.ds(i * 16, 16)] = v
        return carry
    lax.fori_loop(0, n // 16, body, 0)


def _hist_body(dst_h, et_h, out_h, cnt_sh, dbuf, tbuf, segbuf, ones, zbuf):
    c = lax.axis_index("c")
    s = lax.axis_index("s")
    _fill(zbuf, 3200, 0.0)
    _fill(ones, BATCH, 1.0)
    for j in range(8):
        pltpu.sync_copy(zbuf.at[pl.ds(0, 3128)],
                        cnt_sh.at[pl.ds(s * 25024 + j * 3128, 3128)])
    plsc.subcore_barrier()

    def macro_body(k, carry):
        base = s * EP + k * MACRO
        pltpu.sync_copy(dst_h.at[pl.ds(base, MACRO)], dbuf)
        pltpu.sync_copy(et_h.at[pl.ds(base, MACRO)], tbuf)
        for b in range(NBB):
            for j in range(BATCH // 16):
                o = b * BATCH + j * 16
                dv = dbuf[pl.ds(o, 16)]
                tv = tbuf[pl.ds(o, 16)]
                segbuf[pl.ds(j * 16, 16)] = dv * R + tv
            pltpu.sync_copy(ones, cnt_sh.at[segbuf], add=True)
        return carry

    lax.fori_loop(0, NB, macro_body, 0)
    plsc.subcore_barrier()
    half = NRP // 2
    for j in range(4):
        off = c * half + s * 12512 + j * 3128
        pltpu.sync_copy(cnt_sh.at[pl.ds(off, 3128)], zbuf.at[pl.ds(0, 3128)])
        pltpu.sync_copy(zbuf.at[pl.ds(0, 3128)], out_h.at[pl.ds(off, 3128)])


_hist = functools.partial(
    pl.kernel,
    out_type=jax.ShapeDtypeStruct((NRP,), jnp.float32),
    mesh=_mesh,
    compiler_params=_params,
    scratch_types=[
        pltpu.VMEM_SHARED((NRP,), jnp.float32),
        pltpu.VMEM((MACRO,), jnp.int32),
        pltpu.VMEM((MACRO,), jnp.int32),
        pltpu.VMEM((BATCH,), jnp.int32),
        pltpu.VMEM((BATCH,), jnp.float32),
        pltpu.VMEM((3200,), jnp.float32),
    ],
)(_hist_body)


WEP = E2 // 32
WNB = WEP // MACRO


def _wk_body(dst_h, et_h, cnt_h, out_h, dbuf, tbuf, segbuf, cbuf, wbuf, gsem):
    c = lax.axis_index("c")
    s = lax.axis_index("s")
    wid = c * NS + s

    def macro_body(k, carry):
        base = wid * WEP + k * MACRO
        pltpu.sync_copy(dst_h.at[pl.ds(base, MACRO)], dbuf)
        pltpu.sync_copy(et_h.at[pl.ds(base, MACRO)], tbuf)
        for b in range(NBB):
            for j in range(BATCH // 16):
                o = b * BATCH + j * 16
                dv = dbuf[pl.ds(o, 16)]
                tv = tbuf[pl.ds(o, 16)]
                segbuf[b, pl.ds(j * 16, 16)] = dv * R + tv
        descs = [pltpu.async_copy(cnt_h.at[segbuf.at[b]],
                                  cbuf.at[pl.ds(b * BATCH, BATCH)], gsem)
                 for b in range(NBB)]
        for d in descs:
            d.wait()
        for j in range(MACRO // 16):
            cv = cbuf[pl.ds(j * 16, 16)]
            wbuf[pl.ds(j * 16, 16)] = 1.0 / jnp.maximum(cv, 1.0)
        pltpu.sync_copy(wbuf, out_h.at[pl.ds(base, MACRO)])
        return carry

    lax.fori_loop(0, WNB, macro_body, 0)


_wk = functools.partial(
    pl.kernel,
    out_type=jax.ShapeDtypeStruct((E2,), jnp.float32),
    mesh=_mesh,
    compiler_params=_params,
    scratch_types=[
        pltpu.VMEM((MACRO,), jnp.int32),
        pltpu.VMEM((MACRO,), jnp.int32),
        pltpu.VMEM((NBB, BATCH), jnp.int32),
        pltpu.VMEM((MACRO,), jnp.float32),
        pltpu.VMEM((MACRO,), jnp.float32),
        pltpu.SemaphoreType.DMA,
    ],
)(_wk_body)


def _edge_body(src_h, et_h, dst_h, w_h, htab_h, out_h,
               acc_sh, sbuf, tbuf, dbuf, wsbuf, gbuf, lbuf, wm, rows,
               gsem, stsem, ssem0, ssem1, ssem2, ssem3):
    c = lax.axis_index("c")
    s = lax.axis_index("s")
    ssems = (ssem0, ssem1, ssem2, ssem3)
    tab = htab_h.at[c]

    def zfill(i, carry):
        for kk in range(2):
            rows[i, pl.ds(kk * 16, 16)] = jnp.zeros((16,), jnp.float32)
        return carry
    lax.fori_loop(0, BATCH, zfill, 0)
    for j in range(24):
        pltpu.sync_copy(rows.at[pl.ds(0, 128)],
                        acc_sh.at[pl.ds(s * 3136 + j * 128, 128)])
    pltpu.sync_copy(rows.at[pl.ds(0, 64)],
                    acc_sh.at[pl.ds(s * 3136 + 3072, 64)])
    plsc.subcore_barrier()

    def _stage(m, q, issue):
        base = s * EP + m * MACRO
        f = pltpu.async_copy if issue else pltpu.make_async_copy
        return [
            f(src_h.at[pl.ds(base, MACRO)], sbuf.at[q], stsem),
            f(et_h.at[pl.ds(base, MACRO)], tbuf.at[q], stsem),
            f(dst_h.at[pl.ds(base, MACRO)], dbuf.at[q], stsem),
            f(w_h.at[pl.ds(base, MACRO)], wsbuf.at[q], stsem),
        ]

    _stage(0, 0, True)

    def _macro(mm, mq):
        m = 2 * mm + mq
        for d in _stage(m, mq, False):
            d.wait()
        for b in range(NBB):
            for j in range(BATCH // 16):
                o = b * BATCH + j * 16
                sv = sbuf[mq, pl.ds(o, 16)]
                tv = tbuf[mq, pl.ds(o, 16)]
                dv = dbuf[mq, pl.ds(o, 16)]
                wv = wsbuf[mq, pl.ds(o, 16)]
                gbuf[mq, b, pl.ds(j * 16, 16)] = tv * N + sv
                lbuf[mq, b, pl.ds(j * 16, 16)] = dv
                wm[mq, pl.ds(o, 16)] = jnp.where(dv < N, wv, 0.0)
        gds = []
        for b in range(NBB):
            slot = 2 * mq + b
            sl = slot * BATCH

            @pl.when(m >= 2)
            def _():
                pltpu.make_async_copy(rows.at[pl.ds(sl, BATCH)],
                                      acc_sh.at[lbuf.at[0, 0]],
                                      ssems[slot]).wait()
            gds.append(pltpu.async_copy(tab.at[gbuf.at[mq, b]],
                                        rows.at[pl.ds(sl, BATCH)], gsem))

        @pl.when(m < NB - 1)
        def _():
            _stage(m + 1, 1 - mq, True)

        for b in range(NBB):
            slot = 2 * mq + b
            sl = slot * BATCH
            gds[b].wait()
            wmv = wm.at[mq]

            def mul_body(i, carry2):
                for u in range(4):
                    ri = sl + i * 4 + u
                    wv = plsc.load_gather(
                        wmv, [jnp.full((16,), b * BATCH + i * 4 + u,
                                       jnp.int32)])
                    for kk in range(2):
                        rows[ri, pl.ds(kk * 16, 16)] = (
                            rows[ri, pl.ds(kk * 16, 16)] * wv)
                return carry2
            lax.fori_loop(0, BATCH // 4, mul_body, 0)
            pltpu.async_copy(rows.at[pl.ds(sl, BATCH)],
                             acc_sh.at[lbuf.at[mq, b]], ssems[slot], add=True)

    def mm_body(mm, carry):
        _macro(mm, 0)
        _macro(mm, 1)
        return carry

    lax.fori_loop(0, NB // 2, mm_body, 0)
    for slot in range(4):
        pltpu.make_async_copy(rows.at[pl.ds(slot * BATCH, BATCH)],
                              acc_sh.at[lbuf.at[0, 0]], ssems[slot]).wait()
    plsc.subcore_barrier()

    for j in range(24):
        pltpu.sync_copy(acc_sh.at[pl.ds(s * 3136 + j * 128, 128)],
                        rows.at[pl.ds(0, 128)])
        pltpu.sync_copy(rows.at[pl.ds(0, 128)],
                        out_h.at[c, pl.ds(s * 3136 + j * 128, 128)])
    pltpu.sync_copy(acc_sh.at[pl.ds(s * 3136 + 3072, 64)],
                    rows.at[pl.ds(0, 64)])
    pltpu.sync_copy(rows.at[pl.ds(0, 64)],
                    out_h.at[c, pl.ds(s * 3136 + 3072, 64)])


_edge = functools.partial(
    pl.kernel,
    out_type=jax.ShapeDtypeStruct((2, ACC, HD), jnp.float32),
    mesh=_mesh,
    compiler_params=_params,
    scratch_types=[
        pltpu.VMEM_SHARED((ACC, HD), jnp.float32),
        pltpu.VMEM((2, MACRO), jnp.int32),
        pltpu.VMEM((2, MACRO), jnp.int32),
        pltpu.VMEM((2, MACRO), jnp.int32),
        pltpu.VMEM((2, MACRO), jnp.float32),
        pltpu.VMEM((2, NBB, BATCH), jnp.int32),
        pltpu.VMEM((2, NBB, BATCH), jnp.int32),
        pltpu.VMEM((2, MACRO), jnp.float32),
        pltpu.VMEM((4 * BATCH, HD), jnp.float32),
        pltpu.SemaphoreType.DMA,
        pltpu.SemaphoreType.DMA,
        pltpu.SemaphoreType.DMA,
        pltpu.SemaphoreType.DMA,
        pltpu.SemaphoreType.DMA,
        pltpu.SemaphoreType.DMA,
    ],
)(_edge_body)


def _mm_body(h_ref, w_ref, o_ref):
    h = h_ref[...]
    for r in range(R + 1):
        d = jnp.dot(h, w_ref[r], preferred_element_type=jnp.float32)
        o_ref[0, r] = d[:, :HD]
        o_ref[1, r] = d[:, HD:]


def _mm(h, wall):
    return pl.pallas_call(
        _mm_body,
        grid=(N // BLK,),
        in_specs=[
            pl.BlockSpec((BLK, D), lambda i: (i, 0)),
            pl.BlockSpec((R + 1, D, D), lambda i: (0, 0, 0)),
        ],
        out_specs=pl.BlockSpec((2, R + 1, BLK, HD), lambda i: (0, 0, i, 0)),
        out_shape=jax.ShapeDtypeStruct((2, R + 1, N, HD), jnp.float32),
    )(h, wall)


def _combine_body_relu(a0, a1, r0, r1, b_ref, o_ref):
    o_ref[:, :HD] = jnp.maximum(a0[0] + r0[0, 0] + b_ref[:, :HD], 0.0)
    o_ref[:, HD:] = jnp.maximum(a1[0] + r1[0, 0] + b_ref[:, HD:], 0.0)


def _combine_body(a0, a1, r0, r1, b_ref, o_ref):
    o_ref[:, :HD] = a0[0] + r0[0, 0] + b_ref[:, :HD]
    o_ref[:, HD:] = a1[0] + r1[0, 0] + b_ref[:, HD:]


def _combine(aggp, hall, b, relu):
    body = _combine_body_relu if relu else _combine_body
    return pl.pallas_call(
        body,
        grid=(N // CBLK,),
        in_specs=[
            pl.BlockSpec((1, CBLK, HD), lambda i: (0, i, 0)),
            pl.BlockSpec((1, CBLK, HD), lambda i: (1, i, 0)),
            pl.BlockSpec((1, 1, CBLK, HD), lambda i: (0, R, i, 0)),
            pl.BlockSpec((1, 1, CBLK, HD), lambda i: (1, R, i, 0)),
            pl.BlockSpec((1, D), lambda i: (0, 0)),
        ],
        out_specs=pl.BlockSpec((CBLK, D), lambda i: (i, 0)),
        out_shape=jax.ShapeDtypeStruct((N, D), jnp.float32),
    )(aggp, aggp, hall, hall, b)


def kernel(x, edge_index, edge_type, emb, W1, root1, b1, W2, root2, b2):
    src = edge_index[0].astype(jnp.int32)
    dst = edge_index[1].astype(jnp.int32)
    et = edge_type.astype(jnp.int32)
    pad = E2 - E
    srcp = jnp.concatenate([src, jnp.zeros((pad,), jnp.int32)])
    etp = jnp.concatenate([et, jnp.zeros((pad,), jnp.int32)])
    dstp = jnp.concatenate([dst, jnp.full((pad,), N, jnp.int32)])
    h = jnp.take(emb, x, axis=0)
    cnt = _hist(dstp, etp)
    w = _wk(dstp, etp, cnt)
    for (W, root, b, relu) in ((W1, root1, b1, True), (W2, root2, b2, False)):
        wall = jnp.concatenate([W, root[None]], axis=0)
        hall = _mm(h, wall)
        aggp = _edge(srcp, etp, dstp, w, hall.reshape(2, (R + 1) * N, HD))
        h = _combine(aggp, hall, b.reshape(1, D), relu)
    return h


# speedup vs baseline: 1.3343x; 1.1325x over previous
import functools

import jax
import jax.numpy as jnp
from jax import lax
from jax.experimental import pallas as pl
from jax.experimental.pallas import tpu as pltpu
from jax.experimental.pallas import tpu_sc as plsc

N = 50000
E = 800000
D = 64
HD = D // 2
R = 8

NS = 16
NC = 2
BATCH = 128
NBB = 2
MACRO = BATCH * NBB
NB = 202
EP = NB * MACRO
E2 = EP * NS
NRP = 400384
ACC = 50176
BLK = 1000
CBLK = 200

_mesh = plsc.VectorSubcoreMesh(core_axis_name="c", subcore_axis_name="s")
_params = pltpu.CompilerParams(needs_layout_passes=False,
                               use_tc_tiling_on_sc=False)


def _fill(ref, n, val):
    v = jnp.full((16,), val, ref.dtype)
    def body(i, carry):
        ref[pl.ds(i * 16, 16)] = v
        return carry
    lax.fori_loop(0, n // 16, body, 0)


def _histw_body(dst_h, et_h, out_h, cnt_sh, dbuf, tbuf, segbuf, ones, zbuf,
                cbuf, wbuf, gsem):
    c = lax.axis_index("c")
    s = lax.axis_index("s")
    _fill(zbuf, 3200, 0.0)
    _fill(ones, BATCH, 1.0)
    for j in range(8):
        pltpu.sync_copy(zbuf.at[pl.ds(0, 3128)],
                        cnt_sh.at[pl.ds(s * 25024 + j * 3128, 3128)])
    plsc.subcore_barrier()

    def macro_body(k, carry):
        base = s * EP + k * MACRO
        pltpu.sync_copy(dst_h.at[pl.ds(base, MACRO)], dbuf)
        pltpu.sync_copy(et_h.at[pl.ds(base, MACRO)], tbuf)
        for b in range(NBB):
            for j in range(BATCH // 16):
                o = b * BATCH + j * 16
                dv = dbuf[pl.ds(o, 16)]
                tv = tbuf[pl.ds(o, 16)]
                segbuf[b, pl.ds(j * 16, 16)] = dv * R + tv
            pltpu.sync_copy(ones, cnt_sh.at[segbuf.at[b]], add=True)
        return carry

    lax.fori_loop(0, NB, macro_body, 0)
    plsc.subcore_barrier()

    wid = c * NS + s

    def wmacro_body(k, carry):
        base = wid * WEP + k * MACRO
        pltpu.sync_copy(dst_h.at[pl.ds(base, MACRO)], dbuf)
        pltpu.sync_copy(et_h.at[pl.ds(base, MACRO)], tbuf)
        for b in range(NBB):
            for j in range(BATCH // 16):
                o = b * BATCH + j * 16
                dv = dbuf[pl.ds(o, 16)]
                tv = tbuf[pl.ds(o, 16)]
                segbuf[b, pl.ds(j * 16, 16)] = dv * R + tv
        descs = [pltpu.async_copy(cnt_sh.at[segbuf.at[b]],
                                  cbuf.at[pl.ds(b * BATCH, BATCH)], gsem)
                 for b in range(NBB)]
        for d in descs:
            d.wait()
        for j in range(MACRO // 16):
            cv = cbuf[pl.ds(j * 16, 16)]
            wbuf[pl.ds(j * 16, 16)] = 1.0 / jnp.maximum(cv, 1.0)
        pltpu.sync_copy(wbuf, out_h.at[pl.ds(base, MACRO)])
        return carry

    lax.fori_loop(0, WNB, wmacro_body, 0)


WEP = E2 // 32
WNB = WEP // MACRO

_histw = functools.partial(
    pl.kernel,
    out_type=jax.ShapeDtypeStruct((E2,), jnp.float32),
    mesh=_mesh,
    compiler_params=_params,
    scratch_types=[
        pltpu.VMEM_SHARED((NRP,), jnp.float32),
        pltpu.VMEM((MACRO,), jnp.int32),
        pltpu.VMEM((MACRO,), jnp.int32),
        pltpu.VMEM((NBB, BATCH), jnp.int32),
        pltpu.VMEM((BATCH,), jnp.float32),
        pltpu.VMEM((3200,), jnp.float32),
        pltpu.VMEM((MACRO,), jnp.float32),
        pltpu.VMEM((MACRO,), jnp.float32),
        pltpu.SemaphoreType.DMA,
    ],
)(_histw_body)


def _edge_body(src_h, et_h, dst_h, w_h, htab_h, out_h,
               acc_sh, sbuf, tbuf, dbuf, wsbuf, gbuf, lbuf, wm, rows,
               gsem, stsem, ssem0, ssem1, ssem2, ssem3):
    c = lax.axis_index("c")
    s = lax.axis_index("s")
    ssems = (ssem0, ssem1, ssem2, ssem3)
    tab = htab_h.at[c]

    def zfill(i, carry):
        for kk in range(2):
            rows[i, pl.ds(kk * 16, 16)] = jnp.zeros((16,), jnp.float32)
        return carry
    lax.fori_loop(0, BATCH, zfill, 0)
    for j in range(24):
        pltpu.sync_copy(rows.at[pl.ds(0, 128)],
                        acc_sh.at[pl.ds(s * 3136 + j * 128, 128)])
    pltpu.sync_copy(rows.at[pl.ds(0, 64)],
                    acc_sh.at[pl.ds(s * 3136 + 3072, 64)])
    plsc.subcore_barrier()

    def _stage(m, q, issue):
        base = s * EP + m * MACRO
        f = pltpu.async_copy if issue else pltpu.make_async_copy
        return [
            f(src_h.at[pl.ds(base, MACRO)], sbuf.at[q], stsem),
            f(et_h.at[pl.ds(base, MACRO)], tbuf.at[q], stsem),
            f(dst_h.at[pl.ds(base, MACRO)], dbuf.at[q], stsem),
            f(w_h.at[pl.ds(base, MACRO)], wsbuf.at[q], stsem),
        ]

    _stage(0, 0, True)

    def _macro(mm, mq):
        m = 2 * mm + mq
        for d in _stage(m, mq, False):
            d.wait()
        for b in range(NBB):
            for j in range(BATCH // 16):
                o = b * BATCH + j * 16
                sv = sbuf[mq, pl.ds(o, 16)]
                tv = tbuf[mq, pl.ds(o, 16)]
                dv = dbuf[mq, pl.ds(o, 16)]
                wv = wsbuf[mq, pl.ds(o, 16)]
                gbuf[mq, b, pl.ds(j * 16, 16)] = tv * N + sv
                lbuf[mq, b, pl.ds(j * 16, 16)] = dv
                wm[mq, pl.ds(o, 16)] = jnp.where(dv < N, wv, 0.0)
        gds = []
        for b in range(NBB):
            slot = 2 * mq + b
            sl = slot * BATCH

            @pl.when(m >= 2)
            def _():
                pltpu.make_async_copy(rows.at[pl.ds(sl, BATCH)],
                                      acc_sh.at[lbuf.at[0, 0]],
                                      ssems[slot]).wait()
            gds.append(pltpu.async_copy(tab.at[gbuf.at[mq, b]],
                                        rows.at[pl.ds(sl, BATCH)], gsem))

        @pl.when(m < NB - 1)
        def _():
            _stage(m + 1, 1 - mq, True)

        for b in range(NBB):
            slot = 2 * mq + b
            sl = slot * BATCH
            gds[b].wait()
            wmv = wm.at[mq]

            def mul_body(i, carry2):
                for u in range(4):
                    ri = sl + i * 4 + u
                    wv = plsc.load_gather(
                        wmv, [jnp.full((16,), b * BATCH + i * 4 + u,
                                       jnp.int32)])
                    for kk in range(2):
                        rows[ri, pl.ds(kk * 16, 16)] = (
                            rows[ri, pl.ds(kk * 16, 16)] * wv)
                return carry2
            lax.fori_loop(0, BATCH // 4, mul_body, 0)
            pltpu.async_copy(rows.at[pl.ds(sl, BATCH)],
                             acc_sh.at[lbuf.at[mq, b]], ssems[slot], add=True)

    def mm_body(mm, carry):
        _macro(mm, 0)
        _macro(mm, 1)
        return carry

    lax.fori_loop(0, NB // 2, mm_body, 0)
    for slot in range(4):
        pltpu.make_async_copy(rows.at[pl.ds(slot * BATCH, BATCH)],
                              acc_sh.at[lbuf.at[0, 0]], ssems[slot]).wait()
    plsc.subcore_barrier()

    for j in range(24):
        pltpu.sync_copy(acc_sh.at[pl.ds(s * 3136 + j * 128, 128)],
                        rows.at[pl.ds(0, 128)])
        pltpu.sync_copy(rows.at[pl.ds(0, 128)],
                        out_h.at[c, pl.ds(s * 3136 + j * 128, 128)])
    pltpu.sync_copy(acc_sh.at[pl.ds(s * 3136 + 3072, 64)],
                    rows.at[pl.ds(0, 64)])
    pltpu.sync_copy(rows.at[pl.ds(0, 64)],
                    out_h.at[c, pl.ds(s * 3136 + 3072, 64)])


_edge = functools.partial(
    pl.kernel,
    out_type=jax.ShapeDtypeStruct((2, ACC, HD), jnp.float32),
    mesh=_mesh,
    compiler_params=_params,
    scratch_types=[
        pltpu.VMEM_SHARED((ACC, HD), jnp.float32),
        pltpu.VMEM((2, MACRO), jnp.int32),
        pltpu.VMEM((2, MACRO), jnp.int32),
        pltpu.VMEM((2, MACRO), jnp.int32),
        pltpu.VMEM((2, MACRO), jnp.float32),
        pltpu.VMEM((2, NBB, BATCH), jnp.int32),
        pltpu.VMEM((2, NBB, BATCH), jnp.int32),
        pltpu.VMEM((2, MACRO), jnp.float32),
        pltpu.VMEM((4 * BATCH, HD), jnp.float32),
        pltpu.SemaphoreType.DMA,
        pltpu.SemaphoreType.DMA,
        pltpu.SemaphoreType.DMA,
        pltpu.SemaphoreType.DMA,
        pltpu.SemaphoreType.DMA,
        pltpu.SemaphoreType.DMA,
    ],
)(_edge_body)


def _mm_body(h_ref, w_ref, o_ref):
    h = h_ref[...]
    for r in range(R + 1):
        d = jnp.dot(h, w_ref[r], preferred_element_type=jnp.float32)
        o_ref[0, r] = d[:, :HD]
        o_ref[1, r] = d[:, HD:]


def _mm(h, wall):
    return pl.pallas_call(
        _mm_body,
        grid=(N // BLK,),
        in_specs=[
            pl.BlockSpec((BLK, D), lambda i: (i, 0)),
            pl.BlockSpec((R + 1, D, D), lambda i: (0, 0, 0)),
        ],
        out_specs=pl.BlockSpec((2, R + 1, BLK, HD), lambda i: (0, 0, i, 0)),
        out_shape=jax.ShapeDtypeStruct((2, R + 1, N, HD), jnp.float32),
    )(h, wall)


def _combine_body_relu(a0, a1, r0, r1, b_ref, o_ref):
    o_ref[:, :HD] = jnp.maximum(a0[0] + r0[0, 0] + b_ref[:, :HD], 0.0)
    o_ref[:, HD:] = jnp.maximum(a1[0] + r1[0, 0] + b_ref[:, HD:], 0.0)


def _combine_body(a0, a1, r0, r1, b_ref, o_ref):
    o_ref[:, :HD] = a0[0] + r0[0, 0] + b_ref[:, :HD]
    o_ref[:, HD:] = a1[0] + r1[0, 0] + b_ref[:, HD:]


def _combine(aggp, hall, b, relu):
    body = _combine_body_relu if relu else _combine_body
    return pl.pallas_call(
        body,
        grid=(N // CBLK,),
        in_specs=[
            pl.BlockSpec((1, CBLK, HD), lambda i: (0, i, 0)),
            pl.BlockSpec((1, CBLK, HD), lambda i: (1, i, 0)),
            pl.BlockSpec((1, 1, CBLK, HD), lambda i: (0, R, i, 0)),
            pl.BlockSpec((1, 1, CBLK, HD), lambda i: (1, R, i, 0)),
            pl.BlockSpec((1, D), lambda i: (0, 0)),
        ],
        out_specs=pl.BlockSpec((CBLK, D), lambda i: (i, 0)),
        out_shape=jax.ShapeDtypeStruct((N, D), jnp.float32),
    )(aggp, aggp, hall, hall, b)


def kernel(x, edge_index, edge_type, emb, W1, root1, b1, W2, root2, b2):
    src = edge_index[0].astype(jnp.int32)
    dst = edge_index[1].astype(jnp.int32)
    et = edge_type.astype(jnp.int32)
    pad = E2 - E
    srcp = jnp.concatenate([src, jnp.zeros((pad,), jnp.int32)])
    etp = jnp.concatenate([et, jnp.zeros((pad,), jnp.int32)])
    dstp = jnp.concatenate([dst, jnp.full((pad,), N, jnp.int32)])
    h = jnp.take(emb, x, axis=0)
    w = _histw(dstp, etp)
    for (W, root, b, relu) in ((W1, root1, b1, True), (W2, root2, b2, False)):
        wall = jnp.concatenate([W, root[None]], axis=0)
        hall = _mm(h, wall)
        aggp = _edge(srcp, etp, dstp, w, hall.reshape(2, (R + 1) * N, HD))
        h = _combine(aggp, hall, b.reshape(1, D), relu)
    return h


# speedup vs baseline: 1.6416x; 1.2303x over previous
import functools

import jax
import jax.numpy as jnp
from jax import lax
from jax.experimental import pallas as pl
from jax.experimental.pallas import tpu as pltpu
from jax.experimental.pallas import tpu_sc as plsc

N = 50000
E = 800000
D = 64
HD = D // 2
R = 8

NS = 16
NC = 2
BATCH = 128
NBB = 3
MACRO = BATCH * NBB
NB = 132
EP = NB * MACRO
E2 = EP * NS
NRP = 400384
ACC = 50176
BLK = 1000
CBLK = 200

_mesh = plsc.VectorSubcoreMesh(core_axis_name="c", subcore_axis_name="s")
_params = pltpu.CompilerParams(needs_layout_passes=False,
                               use_tc_tiling_on_sc=False)


def _fill(ref, n, val):
    v = jnp.full((16,), val, ref.dtype)
    def body(i, carry):
        ref[pl.ds(i * 16, 16)] = v
        return carry
    lax.fori_loop(0, n // 16, body, 0)


def _histw_body(dst_h, et_h, out_h, cnt_sh, dbuf, tbuf, segbuf, ones, zbuf,
                cbuf, wbuf, gsem):
    c = lax.axis_index("c")
    s = lax.axis_index("s")
    _fill(zbuf, 3200, 0.0)
    _fill(ones, BATCH, 1.0)
    for j in range(8):
        pltpu.sync_copy(zbuf.at[pl.ds(0, 3128)],
                        cnt_sh.at[pl.ds(s * 25024 + j * 3128, 3128)])
    plsc.subcore_barrier()

    def macro_body(k, carry):
        base = s * EP + k * MACRO
        pltpu.sync_copy(dst_h.at[pl.ds(base, MACRO)], dbuf)
        pltpu.sync_copy(et_h.at[pl.ds(base, MACRO)], tbuf)
        for b in range(NBB):
            for j in range(BATCH // 16):
                o = b * BATCH + j * 16
                dv = dbuf[pl.ds(o, 16)]
                tv = tbuf[pl.ds(o, 16)]
                segbuf[b, pl.ds(j * 16, 16)] = dv * R + tv
            pltpu.sync_copy(ones, cnt_sh.at[segbuf.at[b]], add=True)
        return carry

    lax.fori_loop(0, NB, macro_body, 0)
    plsc.subcore_barrier()

    wid = c * NS + s

    def wmacro_body(k, carry):
        base = wid * WEP + k * MACRO
        pltpu.sync_copy(dst_h.at[pl.ds(base, MACRO)], dbuf)
        pltpu.sync_copy(et_h.at[pl.ds(base, MACRO)], tbuf)
        for b in range(NBB):
            for j in range(BATCH // 16):
                o = b * BATCH + j * 16
                dv = dbuf[pl.ds(o, 16)]
                tv = tbuf[pl.ds(o, 16)]
                segbuf[b, pl.ds(j * 16, 16)] = dv * R + tv
        descs = [pltpu.async_copy(cnt_sh.at[segbuf.at[b]],
                                  cbuf.at[pl.ds(b * BATCH, BATCH)], gsem)
                 for b in range(NBB)]
        for d in descs:
            d.wait()
        for j in range(MACRO // 16):
            cv = cbuf[pl.ds(j * 16, 16)]
            wbuf[pl.ds(j * 16, 16)] = 1.0 / jnp.maximum(cv, 1.0)
        pltpu.sync_copy(wbuf, out_h.at[pl.ds(base, MACRO)])
        return carry

    lax.fori_loop(0, WNB, wmacro_body, 0)


WEP = E2 // 32
WNB = WEP // MACRO

_histw = functools.partial(
    pl.kernel,
    out_type=jax.ShapeDtypeStruct((E2,), jnp.float32),
    mesh=_mesh,
    compiler_params=_params,
    scratch_types=[
        pltpu.VMEM_SHARED((NRP,), jnp.float32),
        pltpu.VMEM((MACRO,), jnp.int32),
        pltpu.VMEM((MACRO,), jnp.int32),
        pltpu.VMEM((NBB, BATCH), jnp.int32),
        pltpu.VMEM((BATCH,), jnp.float32),
        pltpu.VMEM((3200,), jnp.float32),
        pltpu.VMEM((MACRO,), jnp.float32),
        pltpu.VMEM((MACRO,), jnp.float32),
        pltpu.SemaphoreType.DMA,
    ],
)(_histw_body)


def _edge_body(src_h, et_h, dst_h, w_h, htab_h, out_h,
               acc_sh, sbuf, tbuf, dbuf, wsbuf, gbuf, lbuf, wm, rows,
               gsem, stsem, ssem0, ssem1, ssem2, ssem3, ssem4, ssem5):
    c = lax.axis_index("c")
    s = lax.axis_index("s")
    ssems = (ssem0, ssem1, ssem2, ssem3, ssem4, ssem5)
    tab = htab_h.at[c]

    def zfill(i, carry):
        for kk in range(2):
            rows[i, pl.ds(kk * 16, 16)] = jnp.zeros((16,), jnp.float32)
        return carry
    lax.fori_loop(0, BATCH, zfill, 0)
    for j in range(24):
        pltpu.sync_copy(rows.at[pl.ds(0, 128)],
                        acc_sh.at[pl.ds(s * 3136 + j * 128, 128)])
    pltpu.sync_copy(rows.at[pl.ds(0, 64)],
                    acc_sh.at[pl.ds(s * 3136 + 3072, 64)])
    plsc.subcore_barrier()

    def _stage(m, q, issue):
        base = s * EP + m * MACRO
        f = pltpu.async_copy if issue else pltpu.make_async_copy
        return [
            f(src_h.at[pl.ds(base, MACRO)], sbuf.at[q], stsem),
            f(et_h.at[pl.ds(base, MACRO)], tbuf.at[q], stsem),
            f(dst_h.at[pl.ds(base, MACRO)], dbuf.at[q], stsem),
            f(w_h.at[pl.ds(base, MACRO)], wsbuf.at[q], stsem),
        ]

    _stage(0, 0, True)

    def _macro(mm, mq):
        m = 2 * mm + mq
        for d in _stage(m, mq, False):
            d.wait()
        for b in range(NBB):
            for j in range(BATCH // 16):
                o = b * BATCH + j * 16
                sv = sbuf[mq, pl.ds(o, 16)]
                tv = tbuf[mq, pl.ds(o, 16)]
                dv = dbuf[mq, pl.ds(o, 16)]
                wv = wsbuf[mq, pl.ds(o, 16)]
                gbuf[mq, b, pl.ds(j * 16, 16)] = tv * N + sv
                lbuf[mq, b, pl.ds(j * 16, 16)] = dv
                wm[mq, pl.ds(o, 16)] = jnp.where(dv < N, wv, 0.0)
        gds = []
        for b in range(NBB):
            slot = NBB * mq + b
            sl = slot * BATCH

            @pl.when(m >= 2)
            def _():
                pltpu.make_async_copy(rows.at[pl.ds(sl, BATCH)],
                                      acc_sh.at[lbuf.at[0, 0]],
                                      ssems[slot]).wait()
            gds.append(pltpu.async_copy(tab.at[gbuf.at[mq, b]],
                                        rows.at[pl.ds(sl, BATCH)], gsem))

        @pl.when(m < NB - 1)
        def _():
            _stage(m + 1, 1 - mq, True)

        for b in range(NBB):
            slot = NBB * mq + b
            sl = slot * BATCH
            gds[b].wait()
            wmv = wm.at[mq]

            def mul_body(i, carry2):
                for u in range(4):
                    ri = sl + i * 4 + u
                    wv = plsc.load_gather(
                        wmv, [jnp.full((16,), b * BATCH + i * 4 + u,
                                       jnp.int32)])
                    for kk in range(2):
                        rows[ri, pl.ds(kk * 16, 16)] = (
                            rows[ri, pl.ds(kk * 16, 16)] * wv)
                return carry2
            lax.fori_loop(0, BATCH // 4, mul_body, 0)
            pltpu.async_copy(rows.at[pl.ds(sl, BATCH)],
                             acc_sh.at[lbuf.at[mq, b]], ssems[slot], add=True)

    def mm_body(mm, carry):
        _macro(mm, 0)
        _macro(mm, 1)
        return carry

    lax.fori_loop(0, NB // 2, mm_body, 0)
    for slot in range(2 * NBB):
        pltpu.make_async_copy(rows.at[pl.ds(slot * BATCH, BATCH)],
                              acc_sh.at[lbuf.at[0, 0]], ssems[slot]).wait()
    plsc.subcore_barrier()

    for j in range(24):
        pltpu.sync_copy(acc_sh.at[pl.ds(s * 3136 + j * 128, 128)],
                        rows.at[pl.ds(0, 128)])
        pltpu.sync_copy(rows.at[pl.ds(0, 128)],
                        out_h.at[c, pl.ds(s * 3136 + j * 128, 128)])
    pltpu.sync_copy(acc_sh.at[pl.ds(s * 3136 + 3072, 64)],
                    rows.at[pl.ds(0, 64)])
    pltpu.sync_copy(rows.at[pl.ds(0, 64)],
                    out_h.at[c, pl.ds(s * 3136 + 3072, 64)])


_edge = functools.partial(
    pl.kernel,
    out_type=jax.ShapeDtypeStruct((2, ACC, HD), jnp.float32),
    mesh=_mesh,
    compiler_params=_params,
    scratch_types=[
        pltpu.VMEM_SHARED((ACC, HD), jnp.float32),
        pltpu.VMEM((2, MACRO), jnp.int32),
        pltpu.VMEM((2, MACRO), jnp.int32),
        pltpu.VMEM((2, MACRO), jnp.int32),
        pltpu.VMEM((2, MACRO), jnp.float32),
        pltpu.VMEM((2, NBB, BATCH), jnp.int32),
        pltpu.VMEM((2, NBB, BATCH), jnp.int32),
        pltpu.VMEM((2, MACRO), jnp.float32),
        pltpu.VMEM((2 * NBB * BATCH, HD), jnp.float32),
        pltpu.SemaphoreType.DMA,
        pltpu.SemaphoreType.DMA,
        pltpu.SemaphoreType.DMA,
        pltpu.SemaphoreType.DMA,
        pltpu.SemaphoreType.DMA,
        pltpu.SemaphoreType.DMA,
        pltpu.SemaphoreType.DMA,
        pltpu.SemaphoreType.DMA,
    ],
)(_edge_body)


def _mm_body(h_ref, w_ref, o_ref):
    h = h_ref[...]
    for r in range(R + 1):
        d = jnp.dot(h, w_ref[r], preferred_element_type=jnp.float32)
        o_ref[0, r] = d[:, :HD]
        o_ref[1, r] = d[:, HD:]


def _mm(h, wall):
    return pl.pallas_call(
        _mm_body,
        grid=(N // BLK,),
        in_specs=[
            pl.BlockSpec((BLK, D), lambda i: (i, 0)),
            pl.BlockSpec((R + 1, D, D), lambda i: (0, 0, 0)),
        ],
        out_specs=pl.BlockSpec((2, R + 1, BLK, HD), lambda i: (0, 0, i, 0)),
        out_shape=jax.ShapeDtypeStruct((2, R + 1, N, HD), jnp.float32),
    )(h, wall)


def _combine_body_relu(a0, a1, r0, r1, b_ref, o_ref):
    o_ref[:, :HD] = jnp.maximum(a0[0] + r0[0, 0] + b_ref[:, :HD], 0.0)
    o_ref[:, HD:] = jnp.maximum(a1[0] + r1[0, 0] + b_ref[:, HD:], 0.0)


def _combine_body(a0, a1, r0, r1, b_ref, o_ref):
    o_ref[:, :HD] = a0[0] + r0[0, 0] + b_ref[:, :HD]
    o_ref[:, HD:] = a1[0] + r1[0, 0] + b_ref[:, HD:]


def _combine(aggp, hall, b, relu):
    body = _combine_body_relu if relu else _combine_body
    return pl.pallas_call(
        body,
        grid=(N // CBLK,),
        in_specs=[
            pl.BlockSpec((1, CBLK, HD), lambda i: (0, i, 0)),
            pl.BlockSpec((1, CBLK, HD), lambda i: (1, i, 0)),
            pl.BlockSpec((1, 1, CBLK, HD), lambda i: (0, R, i, 0)),
            pl.BlockSpec((1, 1, CBLK, HD), lambda i: (1, R, i, 0)),
            pl.BlockSpec((1, D), lambda i: (0, 0)),
        ],
        out_specs=pl.BlockSpec((CBLK, D), lambda i: (i, 0)),
        out_shape=jax.ShapeDtypeStruct((N, D), jnp.float32),
    )(aggp, aggp, hall, hall, b)


def kernel(x, edge_index, edge_type, emb, W1, root1, b1, W2, root2, b2):
    src = edge_index[0].astype(jnp.int32)
    dst = edge_index[1].astype(jnp.int32)
    et = edge_type.astype(jnp.int32)
    pad = E2 - E
    srcp = jnp.concatenate([src, jnp.zeros((pad,), jnp.int32)])
    etp = jnp.concatenate([et, jnp.zeros((pad,), jnp.int32)])
    dstp = jnp.concatenate([dst, jnp.full((pad,), N, jnp.int32)])
    h = jnp.take(emb, x, axis=0)
    w = _histw(dstp, etp)
    for (W, root, b, relu) in ((W1, root1, b1, True), (W2, root2, b2, False)):
        wall = jnp.concatenate([W, root[None]], axis=0)
        hall = _mm(h, wall)
        aggp = _edge(srcp, etp, dstp, w, hall.reshape(2, (R + 1) * N, HD))
        h = _combine(aggp, hall, b.reshape(1, D), relu)
    return h
